# Initial kernel scaffold; baseline (speedup 1.0000x reference)
import jax
import jax.numpy as jnp
from jax.experimental import pallas as pl


def kernel(cls_indices, seg_score, boxes):
    raise NotImplementedError("write your pallas kernel here")


# speedup vs baseline: 1.1046x; 1.1046x over previous
import jax
import jax.numpy as jnp
from jax.experimental import pallas as pl
from jax.experimental.pallas import tpu as pltpu

NUM_SEG_CLASSES = 19
NUM_STUFF = 11
NUM_BOXES = 200
H, W = 256, 512
BOX_SCALE = 0.25


def _seg_kernel(cls_ref, boxes_ref, seg_ref, stuff_ref, inst_ref):
    n = pl.program_id(0)

    @pl.when(n < NUM_STUFF)
    def _():
        stuff_ref[0, 0] = seg_ref[0, n]

    cls_n = cls_ref[n]
    mapped = jnp.clip(cls_n + 10, 0, NUM_SEG_CLASSES - 1)
    x0 = jnp.floor(boxes_ref[n, 1] * BOX_SCALE).astype(jnp.int32)
    y0 = jnp.floor(boxes_ref[n, 2] * BOX_SCALE).astype(jnp.int32)
    x1 = (jnp.round(boxes_ref[n, 3] * BOX_SCALE) + 1.0).astype(jnp.int32)
    y1 = (jnp.round(boxes_ref[n, 4] * BOX_SCALE) + 1.0).astype(jnp.int32)

    rows = jax.lax.broadcasted_iota(jnp.int32, (H, 1), 0)
    cols = jax.lax.broadcasted_iota(jnp.int32, (1, W), 1)
    row_ok = (rows >= y0) & (rows < y1) & (cls_n != 0)
    col_ok = (cols >= x0) & (cols < x1)
    mask = row_ok & col_ok
    inst_ref[0, 0] = jnp.where(mask, seg_ref[0, mapped], 0.0)


def kernel(cls_indices, seg_score, boxes):
    cls_indices = cls_indices.astype(jnp.int32)
    boxes = boxes.astype(jnp.float32)
    stuff, inst = pl.pallas_call(
        _seg_kernel,
        grid=(NUM_BOXES,),
        in_specs=[
            pl.BlockSpec(memory_space=pltpu.SMEM),
            pl.BlockSpec(memory_space=pltpu.SMEM),
            pl.BlockSpec(
                (1, NUM_SEG_CLASSES, H, W), lambda n: (0, 0, 0, 0)
            ),
        ],
        out_specs=[
            pl.BlockSpec((1, 1, H, W), lambda n: (0, jnp.minimum(n, NUM_STUFF - 1), 0, 0)),
            pl.BlockSpec((1, 1, H, W), lambda n: (0, n, 0, 0)),
        ],
        out_shape=[
            jax.ShapeDtypeStruct((1, NUM_STUFF, H, W), jnp.float32),
            jax.ShapeDtypeStruct((1, NUM_BOXES, H, W), jnp.float32),
        ],
    )(cls_indices, boxes, seg_score)
    return (stuff, inst)
